# baseline (speedup 1.0000x reference)
import functools

import jax
import jax.numpy as jnp
from jax import lax
from jax.experimental import pallas as pl
from jax.experimental.pallas import tpu as pltpu
from jax.experimental.pallas import tpu_sc as plsc

DIM = 2048
NC = 2
NS = 16
CH = 16
TN = 256


def _gather_rows(table, ids_flat):
    ntok = ids_flat.shape[0]
    n_work = NC * NS
    b_per_w = ntok // n_work
    mesh = plsc.VectorSubcoreMesh(core_axis_name="c", subcore_axis_name="s")

    n_chunks = (ntok // (NC * NS)) // CH

    @functools.partial(
        pl.kernel,
        mesh=mesh,
        out_type=jax.ShapeDtypeStruct((ntok, DIM), table.dtype),
        scratch_types=[
            pltpu.VMEM((b_per_w,), jnp.int32),
            pltpu.VMEM((CH, DIM), table.dtype),
            pltpu.VMEM((CH, DIM), table.dtype),
            pltpu.SemaphoreType.DMA,
            pltpu.SemaphoreType.DMA,
        ],
    )
    def gather_kernel(table_hbm, idx_hbm, out_hbm, idx_v, rows_a, rows_b, sem_a, sem_b):
        wid = lax.axis_index("s") * NC + lax.axis_index("c")
        base = wid * b_per_w
        pltpu.sync_copy(idx_hbm.at[pl.ds(base, b_per_w)], idx_v)

        bufs = (rows_a, rows_b)
        sems = (sem_a, sem_b)
        copies = [None] * n_chunks
        copies[0] = pltpu.async_copy(
            table_hbm.at[idx_v.at[pl.ds(0, CH)]], bufs[0], sems[0]
        )
        for j in range(n_chunks):
            if j + 1 < n_chunks:
                copies[j + 1] = pltpu.async_copy(
                    table_hbm.at[idx_v.at[pl.ds((j + 1) * CH, CH)]],
                    bufs[(j + 1) % 2],
                    sems[(j + 1) % 2],
                )
            copies[j].wait()
            pltpu.sync_copy(bufs[j % 2], out_hbm.at[pl.ds(base + j * CH, CH)])

    return gather_kernel(table, ids_flat)


def _project(h, W, b2d):
    ntok, vocab = h.shape[0], W.shape[0]

    def mm_kernel(h_ref, w_ref, b_ref, o_ref):
        o_ref[...] = jax.lax.dot_general(
            h_ref[...], w_ref[...],
            (((1,), (1,)), ((), ())),
            preferred_element_type=jnp.float32,
        ) + b_ref[...]

    return pl.pallas_call(
        mm_kernel,
        grid=(vocab // TN,),
        in_specs=[
            pl.BlockSpec((ntok, DIM), lambda i: (0, 0)),
            pl.BlockSpec((TN, DIM), lambda i: (i, 0)),
            pl.BlockSpec((1, TN), lambda i: (0, i)),
        ],
        out_specs=pl.BlockSpec((ntok, TN), lambda i: (0, i)),
        out_shape=jax.ShapeDtypeStruct((ntok, vocab), jnp.float32),
        compiler_params=pltpu.CompilerParams(
            dimension_semantics=("parallel",),
        ),
    )(h, W, b2d)


def kernel(ids, emb_table, W, b):
    batch, seq = ids.shape
    ntok = batch * seq
    vocab = W.shape[0]
    ids_flat = ids.reshape(ntok).astype(jnp.int32)
    b2d = b.reshape(1, -1)

    h = _gather_rows(emb_table, ids_flat)
    logits = _project(h, W, b2d)
    return logits.reshape(batch, seq, vocab)


# speedup vs baseline: 1.0020x; 1.0020x over previous
import functools

import jax
import jax.numpy as jnp
from jax import lax
from jax.experimental import pallas as pl
from jax.experimental.pallas import tpu as pltpu
from jax.experimental.pallas import tpu_sc as plsc

DIM = 2048
NC = 2
NS = 16
CH = 16
TN = 256


def _gather_rows(table, ids_flat):
    ntok = ids_flat.shape[0]
    n_work = NC * NS
    b_per_w = ntok // n_work
    mesh = plsc.VectorSubcoreMesh(core_axis_name="c", subcore_axis_name="s")

    n_chunks = (ntok // (NC * NS)) // CH

    @functools.partial(
        pl.kernel,
        mesh=mesh,
        out_type=jax.ShapeDtypeStruct((ntok, DIM), table.dtype),
        scratch_types=[
            pltpu.VMEM((b_per_w,), jnp.int32),
            pltpu.VMEM((CH, DIM), table.dtype),
            pltpu.VMEM((CH, DIM), table.dtype),
            pltpu.SemaphoreType.DMA,
            pltpu.SemaphoreType.DMA,
            pltpu.SemaphoreType.DMA,
            pltpu.SemaphoreType.DMA,
        ],
    )
    def gather_kernel(table_hbm, idx_hbm, out_hbm, idx_v, rows_a, rows_b,
                      gsem_a, gsem_b, wsem_a, wsem_b):
        wid = lax.axis_index("s") * NC + lax.axis_index("c")
        base = wid * b_per_w
        pltpu.sync_copy(idx_hbm.at[pl.ds(base, b_per_w)], idx_v)

        bufs = (rows_a, rows_b)
        gsems = (gsem_a, gsem_b)
        wsems = (wsem_a, wsem_b)
        gathers = [None] * n_chunks
        writes = [None] * n_chunks
        gathers[0] = pltpu.async_copy(
            table_hbm.at[idx_v.at[pl.ds(0, CH)]], bufs[0], gsems[0]
        )
        for j in range(n_chunks):
            if j + 1 < n_chunks:
                if j >= 1:
                    writes[j - 1].wait()
                gathers[j + 1] = pltpu.async_copy(
                    table_hbm.at[idx_v.at[pl.ds((j + 1) * CH, CH)]],
                    bufs[(j + 1) % 2],
                    gsems[(j + 1) % 2],
                )
            gathers[j].wait()
            writes[j] = pltpu.async_copy(
                bufs[j % 2], out_hbm.at[pl.ds(base + j * CH, CH)], wsems[j % 2]
            )
        writes[n_chunks - 2].wait()
        writes[n_chunks - 1].wait()

    return gather_kernel(table, ids_flat)


def _project(h, W, b2d):
    ntok, vocab = h.shape[0], W.shape[0]

    def mm_kernel(h_ref, w_ref, b_ref, o_ref):
        o_ref[...] = jax.lax.dot_general(
            h_ref[...], w_ref[...],
            (((1,), (1,)), ((), ())),
            preferred_element_type=jnp.float32,
        ) + b_ref[...]

    return pl.pallas_call(
        mm_kernel,
        grid=(vocab // TN,),
        in_specs=[
            pl.BlockSpec((ntok, DIM), lambda i: (0, 0)),
            pl.BlockSpec((TN, DIM), lambda i: (i, 0)),
            pl.BlockSpec((1, TN), lambda i: (0, i)),
        ],
        out_specs=pl.BlockSpec((ntok, TN), lambda i: (0, i)),
        out_shape=jax.ShapeDtypeStruct((ntok, vocab), jnp.float32),
        compiler_params=pltpu.CompilerParams(
            dimension_semantics=("parallel",),
        ),
    )(h, W, b2d)


def kernel(ids, emb_table, W, b):
    batch, seq = ids.shape
    ntok = batch * seq
    vocab = W.shape[0]
    ids_flat = ids.reshape(ntok).astype(jnp.int32)
    b2d = b.reshape(1, -1)

    h = _gather_rows(emb_table, ids_flat)
    logits = _project(h, W, b2d)
    return logits.reshape(batch, seq, vocab)
